# baseline (speedup 1.0000x reference)
import functools

import jax
import jax.numpy as jnp
from jax import lax
from jax.experimental import pallas as pl
from jax.experimental.pallas import tpu as pltpu
from jax.experimental.pallas import tpu_sc as plsc

T, B, N = 128, 4, 50000
PRED_WINDOW = 64
N_DN, N_MBON = 1024, 96
NC, NS = 2, 16
NW = NC * NS
DN_J = N_DN // (NW // B)
MB_J = 16
MB_GROUPS = N_MBON // MB_J

_mesh = plsc.VectorSubcoreMesh(
    core_axis_name="c", subcore_axis_name="s", num_cores=NC, num_subcores=NS
)


@functools.partial(
    pl.kernel,
    out_type=(
        jax.ShapeDtypeStruct((B * N_DN, T), jnp.float32),
        jax.ShapeDtypeStruct((B * N_MBON, T), jnp.float32),
    ),
    mesh=_mesh,
    scratch_types=[
        pltpu.VMEM((DN_J,), jnp.int32),
        pltpu.VMEM((MB_J,), jnp.int32),
        pltpu.VMEM((DN_J, T), jnp.float32),
        pltpu.VMEM((MB_J, T), jnp.float32),
        pltpu.SemaphoreType.DMA,
        pltpu.SemaphoreType.DMA,
    ],
)
def _sc_gather(table, dn_ids_hbm, mbon_ids_hbm, dn_out, mbon_out,
               didx, midx, dn_rows, mb_rows, sem, sem_mb):
    wid = lax.axis_index("s") * NC + lax.axis_index("c")
    b = wid & 3
    q = wid >> 2
    row_base = b * N
    has_mb = q < MB_GROUPS

    cp_ids = pltpu.async_copy(dn_ids_hbm.at[pl.ds(q * DN_J, DN_J)], didx, sem)

    @pl.when(has_mb)
    def _():
        pltpu.async_copy(mbon_ids_hbm.at[pl.ds(q * MB_J, MB_J)], midx,
                         sem_mb).wait()
        midx[pl.ds(0, 16)] = midx[pl.ds(0, 16)] + row_base
    cp_ids.wait()

    def _mk_dn(i, _):
        didx[pl.ds(i * 16, 16)] = didx[pl.ds(i * 16, 16)] + row_base
        return _
    lax.fori_loop(0, DN_J // 16, _mk_dn, None)

    cp_dn = pltpu.async_copy(table.at[didx], dn_rows, sem)

    @pl.when(has_mb)
    def _():
        pltpu.async_copy(table.at[midx], mb_rows, sem_mb).wait()
        pltpu.async_copy(
            mb_rows, mbon_out.at[pl.ds(b * N_MBON + q * MB_J, MB_J)], sem_mb)

    cp_dn.wait()
    pltpu.sync_copy(dn_rows, dn_out.at[pl.ds(b * N_DN + q * DN_J, DN_J)])

    @pl.when(has_mb)
    def _():
        pltpu.make_async_copy(
            mb_rows, mbon_out.at[pl.ds(b * N_MBON + q * MB_J, MB_J)],
            sem_mb).wait()


def kernel(neuron_v, neuron_spike, dn_ids, mbon_ids):
    del neuron_spike
    table = jnp.transpose(neuron_v, (1, 2, 0)).reshape(B * N, T)
    dn_flat, mbon_flat = _sc_gather(table, dn_ids, mbon_ids)
    dn = jnp.transpose(
        dn_flat.reshape(B, N_DN, T)[:, :, T - PRED_WINDOW:], (2, 0, 1))
    mbon = jnp.transpose(
        mbon_flat.reshape(B, N_MBON, T)[:, :, T - PRED_WINDOW:], (2, 0, 1))
    return dn, mbon
